# baseline (speedup 1.0000x reference)
import functools

import jax
import jax.numpy as jnp
from jax import lax
from jax.experimental import pallas as pl
from jax.experimental.pallas import tpu as pltpu
from jax.experimental.pallas import tpu_sc as plsc

_BATCH = 16384
_HIST = 50
_D = 32
_B = _BATCH * _HIST

_info = plsc.get_sparse_core_info()
_NC, _NS = _info.num_cores, _info.num_subcores
_NW = _NC * _NS
_BATCH_PW = _BATCH // _NW
_BPW = _B // _NW
_CB = _BATCH_PW
_NV = _CB // 16
_NBUF = 2


def _make_gather():
  mesh = plsc.VectorSubcoreMesh(core_axis_name="c", subcore_axis_name="s")

  @functools.partial(
      pl.kernel,
      mesh=mesh,
      out_type=jax.ShapeDtypeStruct((_HIST * _D, _BATCH), jnp.float32),
      scratch_types=[
          pltpu.VMEM((_BPW,), jnp.int32),
          pltpu.VMEM((_BPW,), jnp.int32),
          pltpu.VMEM((_NBUF, _CB, _D), jnp.float32),
          pltpu.VMEM((_NBUF, _D, _CB + 1), jnp.float32),
          pltpu.SemaphoreType.DMA,
          pltpu.SemaphoreType.DMA,
          pltpu.SemaphoreType.DMA,
          pltpu.SemaphoreType.DMA,
      ],
      compiler_params=pltpu.CompilerParams(
          use_tc_tiling_on_sc=False, needs_layout_passes=False),
  )
  def gather_kernel(table_hbm, idx_hbm, out_hbm, idx_v, cidx_v, rows_v,
                    tout_v, gs0, gs1, ss0, ss1):
    gsems = (gs0, gs1)
    ssems = (ss0, ss1)
    wid = lax.axis_index("s") * _NC + lax.axis_index("c")
    base = wid * _BPW
    bbase = wid * _BATCH_PW

    lane = lax.iota(jnp.int32, 16)
    lane16 = lane + 16
    lane_h = lane * _HIST

    def gather_copy(t, b):
      return pltpu.make_async_copy(
          table_hbm.at[cidx_v.at[pl.ds(t * _CB, _CB)]],
          rows_v.at[b], gsems[b])

    def store_copy(t, b):
      return pltpu.make_async_copy(
          tout_v.at[b].at[:, pl.ds(0, _CB)],
          out_hbm.at[pl.ds(t * _D, _D), pl.ds(bbase, _CB)], ssems[b])

    pltpu.sync_copy(idx_hbm.at[pl.ds(base, _BPW)], idx_v)

    @plsc.parallel_loop(0, _HIST * (_NV // 8))
    def _(k):
      t = k >> 2
      vg = k & 3
      vt = lane_h + t
      vals = []
      for j in range(8):
        v = vg * 8 + j
        vals.append(
            plsc.load_gather(idx_v.at[pl.ds(v * 16 * _HIST, 16 * _HIST)],
                             [vt]))
      for j in range(8):
        cidx_v[pl.ds(t * _CB + (vg * 8 + j) * 16, 16)] = vals[j]

    gather_copy(0, 0).start()
    gather_copy(1, 1).start()

    def body(t, b):
      gather_copy(t, b).wait()

      @pl.when(t >= _NBUF)
      def _():
        store_copy(t - _NBUF, b).wait()

      rows = rows_v.at[b]
      tout = tout_v.at[b]

      @plsc.parallel_loop(0, _CB, unroll=4)
      def _(j):
        v0 = rows[j, pl.ds(0, 16)]
        v1 = rows[j, pl.ds(16, 16)]
        cj = jnp.zeros((16,), jnp.int32) + j
        plsc.store_scatter(tout, [lane, cj], v0)
        plsc.store_scatter(tout, [lane16, cj], v1)

      store_copy(t, b).start()

      @pl.when(t + _NBUF < _HIST)
      def _():
        gather_copy(t + _NBUF, b).start()

    def outer(g, carry):
      for b in range(_NBUF):
        body(g * _NBUF + b, b)
      return carry

    lax.fori_loop(0, _HIST // _NBUF, outer, 0, unroll=False)
    store_copy(_HIST - 2, 0).wait()
    store_copy(_HIST - 1, 1).wait()

  return gather_kernel


_gather = _make_gather()

_VOCAB = 1000000
_TC = 800
_TCHUNKS = _VOCAB // _TC


def _make_transpose():
  mesh = plsc.VectorSubcoreMesh(core_axis_name="c", subcore_axis_name="s")

  @functools.partial(
      pl.kernel,
      mesh=mesh,
      out_type=jax.ShapeDtypeStruct((_VOCAB, _D), jnp.float32),
      scratch_types=[
          pltpu.VMEM((_NBUF, _D, _TC + 1), jnp.float32),
          pltpu.VMEM((_NBUF, _TC, _D), jnp.float32),
          pltpu.SemaphoreType.DMA,
          pltpu.SemaphoreType.DMA,
          pltpu.SemaphoreType.DMA,
          pltpu.SemaphoreType.DMA,
      ],
      compiler_params=pltpu.CompilerParams(
          use_tc_tiling_on_sc=False, needs_layout_passes=False),
  )
  def transpose_kernel(tt_hbm, out_hbm, tbuf_v, obuf_v, ls0, ls1, ss0, ss1):
    lsems = (ls0, ls1)
    ssems = (ss0, ss1)
    wid = lax.axis_index("s") * _NC + lax.axis_index("c")

    lane = lax.iota(jnp.int32, 16)
    lane16 = lane + 16

    def load_copies(k, b):
      cid = wid + _NW * k
      return [
          pltpu.make_async_copy(
              tt_hbm.at[pl.ds(d * _VOCAB + cid * _TC, _TC)],
              tbuf_v.at[b].at[d, pl.ds(0, _TC)], lsems[b])
          for d in range(_D)
      ]

    def load_start(k, b):
      for c in load_copies(k, b):
        c.start()

    def load_wait(k, b):
      for c in load_copies(k, b):
        c.wait()

    def store_copy(k, b):
      cid = wid + _NW * k
      return pltpu.make_async_copy(
          obuf_v.at[b], out_hbm.at[pl.ds(cid * _TC, _TC), :], ssems[b])

    nk = (_TCHUNKS + _NW - 1) // _NW

    def in_range(k):
      return wid + _NW * k < _TCHUNKS

    @pl.when(in_range(0))
    def _():
      load_start(0, 0)

    @pl.when(in_range(1))
    def _():
      load_start(1, 1)

    def body(k, b):
      @pl.when(in_range(k))
      def _():
        load_wait(k, b)

        @pl.when(k >= _NBUF)
        def _():
          store_copy(k - _NBUF, b).wait()

        tbuf = tbuf_v.at[b]
        obuf = obuf_v.at[b]

        @plsc.parallel_loop(0, _TC, unroll=4)
        def _(j):
          cj = jnp.zeros((16,), jnp.int32) + j
          v0 = plsc.load_gather(tbuf, [lane, cj])
          v1 = plsc.load_gather(tbuf, [lane16, cj])
          obuf[j, pl.ds(0, 16)] = v0
          obuf[j, pl.ds(16, 16)] = v1

        store_copy(k, b).start()

        @pl.when(in_range(k + _NBUF))
        def _():
          load_start(k + _NBUF, b)

    def outer(g, carry):
      for b in range(_NBUF):
        body(g * _NBUF + b, b)
      return carry

    lax.fori_loop(0, nk // _NBUF, outer, 0, unroll=False)

    @pl.when(in_range(nk - 2))
    def _():
      store_copy(nk - 2, 0).wait()

    @pl.when(in_range(nk - 1))
    def _():
      store_copy(nk - 1, 1).wait()

  return transpose_kernel


_transpose = _make_transpose()


def kernel(x, table):
  idx = x.reshape(_B)
  table_rm = _transpose(table.T.reshape(_VOCAB * _D))
  out2d = _gather(table_rm, idx)
  out = out2d.reshape(_HIST, _D, _BATCH).transpose(2, 0, 1)
  return out


# speedup vs baseline: 4.0486x; 4.0486x over previous
import functools

import jax
import jax.numpy as jnp
from jax import lax
from jax.experimental import pallas as pl
from jax.experimental.pallas import tpu as pltpu
from jax.experimental.pallas import tpu_sc as plsc

_BATCH = 16384
_HIST = 50
_D = 32
_B = _BATCH * _HIST

_info = plsc.get_sparse_core_info()
_NC, _NS = _info.num_cores, _info.num_subcores
_NW = _NC * _NS
_BATCH_PW = _BATCH // _NW
_BPW = _B // _NW
_CB = _BATCH_PW
_NV = _CB // 16
_NBUF = 2


def _make_gather():
  mesh = plsc.VectorSubcoreMesh(core_axis_name="c", subcore_axis_name="s")

  @functools.partial(
      pl.kernel,
      mesh=mesh,
      out_type=jax.ShapeDtypeStruct((_HIST * _D, _BATCH), jnp.float32),
      scratch_types=[
          pltpu.VMEM((_BPW,), jnp.int32),
          pltpu.VMEM((_BPW,), jnp.int32),
          pltpu.VMEM((_NBUF, _CB, _D), jnp.float32),
          pltpu.VMEM((_NBUF, _D, _CB + 1), jnp.float32),
          pltpu.SemaphoreType.DMA,
          pltpu.SemaphoreType.DMA,
          pltpu.SemaphoreType.DMA,
          pltpu.SemaphoreType.DMA,
      ],
      compiler_params=pltpu.CompilerParams(
          use_tc_tiling_on_sc=False, needs_layout_passes=False),
  )
  def gather_kernel(table_hbm, idx_hbm, out_hbm, idx_v, cidx_v, rows_v,
                    tout_v, gs0, gs1, ss0, ss1):
    gsems = (gs0, gs1)
    ssems = (ss0, ss1)
    wid = lax.axis_index("s") * _NC + lax.axis_index("c")
    base = wid * _BPW
    bbase = wid * _BATCH_PW

    lane = lax.iota(jnp.int32, 16)
    lane16 = lane + 16
    lane_h = lane * _HIST

    def gather_copy(t, b):
      return pltpu.make_async_copy(
          table_hbm.at[cidx_v.at[pl.ds(t * _CB, _CB)]],
          rows_v.at[b], gsems[b])

    def store_copy(t, b):
      return pltpu.make_async_copy(
          tout_v.at[b].at[:, pl.ds(0, _CB)],
          out_hbm.at[pl.ds(t * _D, _D), pl.ds(bbase, _CB)], ssems[b])

    pltpu.sync_copy(idx_hbm.at[pl.ds(base, _BPW)], idx_v)

    @plsc.parallel_loop(0, _HIST * (_NV // 8))
    def _(k):
      t = k >> 2
      vg = k & 3
      vt = lane_h + t
      vals = []
      for j in range(8):
        v = vg * 8 + j
        vals.append(
            plsc.load_gather(idx_v.at[pl.ds(v * 16 * _HIST, 16 * _HIST)],
                             [vt]))
      for j in range(8):
        cidx_v[pl.ds(t * _CB + (vg * 8 + j) * 16, 16)] = vals[j]

    gather_copy(0, 0).start()
    gather_copy(1, 1).start()

    def body(t, b):
      gather_copy(t, b).wait()

      @pl.when(t >= _NBUF)
      def _():
        store_copy(t - _NBUF, b).wait()

      rows = rows_v.at[b]
      tout = tout_v.at[b]

      @plsc.parallel_loop(0, _CB, unroll=4)
      def _(j):
        v0 = rows[j, pl.ds(0, 16)]
        v1 = rows[j, pl.ds(16, 16)]
        cj = jnp.zeros((16,), jnp.int32) + j
        plsc.store_scatter(tout, [lane, cj], v0)
        plsc.store_scatter(tout, [lane16, cj], v1)

      store_copy(t, b).start()

      @pl.when(t + _NBUF < _HIST)
      def _():
        gather_copy(t + _NBUF, b).start()

    def outer(g, carry):
      for b in range(_NBUF):
        body(g * _NBUF + b, b)
      return carry

    lax.fori_loop(0, _HIST // _NBUF, outer, 0, unroll=False)
    store_copy(_HIST - 2, 0).wait()
    store_copy(_HIST - 1, 1).wait()

  return gather_kernel


_gather = _make_gather()


def kernel(x, table):
  idx = x.reshape(_B)
  out2d = _gather(table, idx)
  out = out2d.reshape(_HIST, _D, _BATCH).transpose(2, 0, 1)
  return out
